# Initial kernel scaffold; baseline (speedup 1.0000x reference)
import jax
import jax.numpy as jnp
from jax.experimental import pallas as pl


def kernel(us, them, white_indices, white_values, black_indices, black_values, psqt_indices, layer_stack_indices, ft_W, ft_b, router_W, router_b, router_ls, W1, b1, W2, b2, W3, b3):
    raise NotImplementedError("write your pallas kernel here")


# speedup vs baseline: 1.5419x; 1.5419x over previous
import functools

import jax
import jax.numpy as jnp
from jax import lax
from jax.experimental import pallas as pl
from jax.experimental.pallas import tpu as pltpu, tpu_sc as plsc

L1 = 1024
NPSQT = 8
NLS = 8
NRF = 16
TAU = 1.0
MAX_FT_ACT = 1.0
L0_CORR = 127.0 / 128.0

D = L1 + NPSQT
D_PAD = 1152
K = 32
NW = 32


def _ft_bag_kernel(table_hbm, idx_hbm, val_hbm, out_hbm,
                   idx_v, val_v, rows0, rows1, ostage0, ostage1,
                   sem_in0, sem_in1, sem_out0, sem_out1):
    nb = idx_hbm.shape[0]
    epw = nb // NW
    wid = lax.axis_index("c") * 16 + lax.axis_index("s")
    base = wid * epw

    pltpu.sync_copy(idx_hbm.at[pl.ds(base, epw)], idx_v)
    pltpu.sync_copy(val_hbm.at[pl.ds(base * K, epw * K)], val_v)

    rows = (rows0, rows1)
    ostage = (ostage0, ostage1)
    sems_in = (sem_in0, sem_in1)
    sems_out = (sem_out0, sem_out1)

    def start_gather(e, slot):
        pltpu.make_async_copy(
            table_hbm.at[idx_v.at[e]], rows[slot], sems_in[slot]
        ).start()

    def wait_gather(e, slot):
        pltpu.make_async_copy(
            table_hbm.at[idx_v.at[e]], rows[slot], sems_in[slot]
        ).wait()

    start_gather(0, 0)
    start_gather(1, 1)

    def do_example(g, slot):
        e = g * 2 + slot
        wait_gather(e, slot)
        vv0 = val_v[pl.ds(pl.multiple_of(e * K, 16), 16)]
        vv1 = val_v[pl.ds(pl.multiple_of(e * K + 16, 16), 16)]
        vb = [jnp.full((16,), vv0[k] if k < 16 else vv1[k - 16], jnp.float32)
              for k in range(K)]

        def chunk_body(c, _):
            off = pl.multiple_of(c * 16, 16)
            acc = rows[slot][0, pl.ds(off, 16)] * vb[0]
            for k in range(1, K):
                acc = acc + rows[slot][k, pl.ds(off, 16)] * vb[k]
            ostage[slot][pl.ds(off, 16)] = acc
            return 0

        @pl.when(g > 0)
        def _():
            pltpu.make_async_copy(
                ostage[slot].at[pl.ds(0, D)], out_hbm.at[pl.ds((base + e - 2) * D, D)],
                sems_out[slot],
            ).wait()

        lax.fori_loop(0, 65, chunk_body, 0)

        pltpu.make_async_copy(
            ostage[slot].at[pl.ds(0, D)], out_hbm.at[pl.ds((base + e) * D, D)],
            sems_out[slot]
        ).start()

        @pl.when(e + 2 < epw)
        def _():
            start_gather(e + 2, slot)

    def outer(g, _):
        do_example(g, 0)
        do_example(g, 1)
        return 0

    lax.fori_loop(0, epw // 2, outer, 0)

    pltpu.make_async_copy(
        ostage0.at[pl.ds(0, D)], out_hbm.at[pl.ds((base + epw - 2) * D, D)],
        sem_out0
    ).wait()
    pltpu.make_async_copy(
        ostage1.at[pl.ds(0, D)], out_hbm.at[pl.ds((base + epw - 1) * D, D)],
        sem_out1
    ).wait()


def _ft_bag(ft_W_pad, idx_all, val_all):
    nb = idx_all.shape[0]
    mesh = plsc.VectorSubcoreMesh(core_axis_name="c", subcore_axis_name="s")
    epw = nb // NW
    return pl.kernel(
        _ft_bag_kernel,
        out_type=jax.ShapeDtypeStruct((nb * D,), jnp.float32),
        mesh=mesh,
        scratch_types=[
            pltpu.VMEM((epw, K), jnp.int32),
            pltpu.VMEM((epw * K,), jnp.float32),
            pltpu.VMEM((K, D_PAD), jnp.float32),
            pltpu.VMEM((K, D_PAD), jnp.float32),
            pltpu.VMEM((1040,), jnp.float32),
            pltpu.VMEM((1040,), jnp.float32),
            pltpu.SemaphoreType.DMA,
            pltpu.SemaphoreType.DMA,
            pltpu.SemaphoreType.DMA,
            pltpu.SemaphoreType.DMA,
        ],
    )(ft_W_pad, idx_all, val_all.reshape(-1))


def _dense_kernel(accw, accb, us, them, g, ftb, rW, rb, rls,
                  W1T, b1f, W2bd, b2f, W3bd, b3f, out):
    wp = accw[...] + ftb[...]
    bp = accb[...] + ftb[...]
    w = wp[:, :L1]
    wps = wp[:, L1:]
    b_ = bp[:, :L1]
    bps = bp[:, L1:]
    u = us[...]
    t = them[...]
    l0w = jnp.clip(u * w + t * b_, 0.0, MAX_FT_ACT)
    l0b = jnp.clip(u * b_ + t * w, 0.0, MAX_FT_ACT)
    half = L1 // 2
    p0 = l0w[:, :half] * l0w[:, half:]
    p1 = l0b[:, :half] * l0b[:, half:]
    l0_ = jnp.concatenate([p0, p1], axis=1) * L0_CORR
    rf = jnp.concatenate([p0[:, half - NRF:], p1[:, half - NRF:]], axis=1)
    logits = rls[0, 0] * (
        jnp.dot(rf, rW[...], preferred_element_type=jnp.float32) + rb[...]
    )
    z = logits + g[...]
    zmax = jnp.max(z, axis=1, keepdims=True)
    iota8 = lax.broadcasted_iota(jnp.int32, z.shape, 1)
    first = jnp.min(jnp.where(z >= zmax, iota8, NLS), axis=1, keepdims=True)
    rw = (iota8 == first).astype(jnp.float32)
    h1 = jnp.clip(
        jnp.dot(l0_, W1T[...], preferred_element_type=jnp.float32) + b1f[...],
        0.0, 1.0)
    h2 = jnp.clip(
        jnp.dot(h1, W2bd[...], preferred_element_type=jnp.float32) + b2f[...],
        0.0, 1.0)
    oe = jnp.dot(h2, W3bd[...], preferred_element_type=jnp.float32) + b3f[...]
    x = jnp.sum(oe * rw, axis=1, keepdims=True)
    psqt = jnp.sum((wps - bps) * rw, axis=1, keepdims=True)
    out[...] = x + psqt * (u - 0.5)


def kernel(us, them, white_indices, white_values, black_indices, black_values,
           psqt_indices, layer_stack_indices, ft_W, ft_b, router_W, router_b,
           router_ls, W1, b1, W2, b2, W3, b3):
    B = us.shape[0]
    idx_all = jnp.concatenate([white_indices, black_indices], axis=0)
    val_all = jnp.concatenate([white_values, black_values], axis=0)

    ft_W_pad = jnp.pad(ft_W, ((0, 0), (0, D_PAD - D)))
    acc = _ft_bag(ft_W_pad, idx_all.astype(jnp.int32),
                  val_all).reshape(2 * B, D)

    u = jax.random.uniform(jax.random.key(42), (B, NLS),
                           minval=1e-6, maxval=1.0 - 1e-6)
    gnoise = -jnp.log(-jnp.log(u)) / TAU

    L2d = W2.shape[1]
    W1T = W1.reshape(NLS * W1.shape[1], L1).T
    b1f = b1.reshape(1, -1)
    e_ids = jnp.arange(NLS)
    W2bd = jnp.zeros((NLS * W2.shape[2], NLS * L2d), jnp.float32)
    W2bd = W2bd.at[
        (e_ids[:, None, None] * W2.shape[2]
         + jnp.arange(W2.shape[2])[None, :, None]),
        (e_ids[:, None, None] * L2d + jnp.arange(L2d)[None, None, :]),
    ].set(jnp.transpose(W2, (0, 2, 1)))
    b2f = b2.reshape(1, -1)
    W3bd = jnp.zeros((NLS * L2d, NLS), jnp.float32)
    W3bd = W3bd.at[
        (e_ids[:, None] * L2d + jnp.arange(L2d)[None, :]),
        e_ids[:, None],
    ].set(W3[:, 0, :])
    b3f = b3.reshape(1, -1)

    BLK = 512
    nblk = B // BLK
    grid = (nblk,)
    z2 = lambda i: (i, 0)
    out = pl.pallas_call(
        _dense_kernel,
        grid=grid,
        in_specs=[
            pl.BlockSpec((BLK, D), z2),
            pl.BlockSpec((BLK, D), lambda i: (i + nblk, 0)),
            pl.BlockSpec((BLK, 1), z2),
            pl.BlockSpec((BLK, 1), z2),
            pl.BlockSpec((BLK, NLS), z2),
            pl.BlockSpec((1, D), lambda i: (0, 0)),
            pl.BlockSpec((2 * NRF, NLS), lambda i: (0, 0)),
            pl.BlockSpec((1, NLS), lambda i: (0, 0)),
            pl.BlockSpec((1, 1), lambda i: (0, 0)),
            pl.BlockSpec((L1, NLS * 16), lambda i: (0, 0)),
            pl.BlockSpec((1, NLS * 16), lambda i: (0, 0)),
            pl.BlockSpec((NLS * 16, NLS * 32), lambda i: (0, 0)),
            pl.BlockSpec((1, NLS * 32), lambda i: (0, 0)),
            pl.BlockSpec((NLS * 32, NLS), lambda i: (0, 0)),
            pl.BlockSpec((1, NLS), lambda i: (0, 0)),
        ],
        out_specs=pl.BlockSpec((BLK, 1), z2),
        out_shape=jax.ShapeDtypeStruct((B, 1), jnp.float32),
    )(acc, acc, us, them, gnoise, ft_b.reshape(1, D), router_W,
      router_b.reshape(1, NLS), router_ls.reshape(1, 1), W1T, b1f,
      W2bd, b2f, W3bd, b3f)
    return out


# speedup vs baseline: 1.9841x; 1.2868x over previous
import functools

import jax
import jax.numpy as jnp
from jax import lax
from jax.experimental import pallas as pl
from jax.experimental.pallas import tpu as pltpu, tpu_sc as plsc

L1 = 1024
NPSQT = 8
NLS = 8
NRF = 16
TAU = 1.0
MAX_FT_ACT = 1.0
L0_CORR = 127.0 / 128.0

D = L1 + NPSQT
K = 32
NW = 32
NBUF = 3


def _ft_bag_kernel(table_hbm, idx_hbm, val_hbm, out_hbm,
                   idx_v, val_v, rows0, rows1, rows2,
                   ostage0, ostage1, ostage2,
                   si0, si1, si2, so0, so1, so2):
    nb = idx_hbm.shape[0]
    epw = nb // NW
    wid = lax.axis_index("c") * 16 + lax.axis_index("s")
    base = wid * epw

    pltpu.sync_copy(idx_hbm.at[pl.ds(base, epw)], idx_v)
    pltpu.sync_copy(val_hbm.at[pl.ds(base * K, epw * K)], val_v)

    rows = (rows0, rows1, rows2)
    ostage = (ostage0, ostage1, ostage2)
    sems_in = (si0, si1, si2)
    sems_out = (so0, so1, so2)

    def start_gather(e, slot):
        pltpu.make_async_copy(
            table_hbm.at[idx_v.at[e]], rows[slot], sems_in[slot]
        ).start()

    def wait_gather(e, slot):
        pltpu.make_async_copy(
            table_hbm.at[idx_v.at[e]], rows[slot], sems_in[slot]
        ).wait()

    def out_copy(e, slot):
        return pltpu.make_async_copy(
            ostage[slot], out_hbm.at[pl.ds((base + e) * D, D)], sems_out[slot]
        )

    for s in range(NBUF):
        start_gather(s, s)

    def do_example(e, slot, may_wait_out):
        wait_gather(e, slot)
        vv0 = val_v[pl.ds(pl.multiple_of(e * K, 16), 16)]
        vv1 = val_v[pl.ds(pl.multiple_of(e * K + 16, 16), 16)]
        vb = [jnp.full((16,), vv0[k] if k < 16 else vv1[k - 16], jnp.float32)
              for k in range(K)]

        def accum(off):
            acc = rows[slot][0, pl.ds(off, 16)] * vb[0]
            for k in range(1, K):
                acc = acc + rows[slot][k, pl.ds(off, 16)] * vb[k]
            ostage[slot][pl.ds(off, 16)] = acc

        def chunk_body(c, _):
            for j in range(4):
                accum(pl.multiple_of(c * 64 + j * 16, 16))
            return 0

        if may_wait_out:
            @pl.when(e >= NBUF)
            def _():
                out_copy(e - NBUF, slot).wait()

        lax.fori_loop(0, 16, chunk_body, 0)
        accum(D - 16)

        out_copy(e, slot).start()

        @pl.when(e + NBUF < epw)
        def _():
            start_gather(e + NBUF, slot)

    def outer(g, _):
        e0 = g * NBUF
        for s in range(NBUF):
            do_example(e0 + s, s, True)
        return 0

    lax.fori_loop(0, (epw - 1) // NBUF, outer, 0)
    do_example(epw - 1, 0, True)

    out_copy(epw - 3, 1).wait()
    out_copy(epw - 2, 2).wait()
    out_copy(epw - 1, 0).wait()


def _ft_bag(ft_W, idx_all, val_all):
    nb = idx_all.shape[0]
    mesh = plsc.VectorSubcoreMesh(core_axis_name="c", subcore_axis_name="s")
    epw = nb // NW
    return pl.kernel(
        _ft_bag_kernel,
        out_type=jax.ShapeDtypeStruct((nb * D,), jnp.float32),
        mesh=mesh,
        compiler_params=pltpu.CompilerParams(use_tc_tiling_on_sc=False),
        scratch_types=[
            pltpu.VMEM((epw, K), jnp.int32),
            pltpu.VMEM((epw * K,), jnp.float32),
            pltpu.VMEM((K, D), jnp.float32),
            pltpu.VMEM((K, D), jnp.float32),
            pltpu.VMEM((K, D), jnp.float32),
            pltpu.VMEM((D,), jnp.float32),
            pltpu.VMEM((D,), jnp.float32),
            pltpu.VMEM((D,), jnp.float32),
            pltpu.SemaphoreType.DMA,
            pltpu.SemaphoreType.DMA,
            pltpu.SemaphoreType.DMA,
            pltpu.SemaphoreType.DMA,
            pltpu.SemaphoreType.DMA,
            pltpu.SemaphoreType.DMA,
        ],
    )(ft_W, idx_all, val_all.reshape(-1))


def _dense_kernel(accw, accb, us, them, g, ftb, rW, rb, rls,
                  W1T, b1f, W2bd, b2f, W3bd, b3f, out):
    wp = accw[...] + ftb[...]
    bp = accb[...] + ftb[...]
    w = wp[:, :L1]
    wps = wp[:, L1:]
    b_ = bp[:, :L1]
    bps = bp[:, L1:]
    u = us[...]
    t = them[...]
    l0w = jnp.clip(u * w + t * b_, 0.0, MAX_FT_ACT)
    l0b = jnp.clip(u * b_ + t * w, 0.0, MAX_FT_ACT)
    half = L1 // 2
    p0 = l0w[:, :half] * l0w[:, half:]
    p1 = l0b[:, :half] * l0b[:, half:]
    l0_ = jnp.concatenate([p0, p1], axis=1) * L0_CORR
    rf = jnp.concatenate([p0[:, half - NRF:], p1[:, half - NRF:]], axis=1)
    logits = rls[0, 0] * (
        jnp.dot(rf, rW[...], preferred_element_type=jnp.float32) + rb[...]
    )
    z = logits + g[...]
    zmax = jnp.max(z, axis=1, keepdims=True)
    iota8 = lax.broadcasted_iota(jnp.int32, z.shape, 1)
    first = jnp.min(jnp.where(z >= zmax, iota8, NLS), axis=1, keepdims=True)
    rw = (iota8 == first).astype(jnp.float32)
    h1 = jnp.clip(
        jnp.dot(l0_, W1T[...], preferred_element_type=jnp.float32) + b1f[...],
        0.0, 1.0)
    h2 = jnp.clip(
        jnp.dot(h1, W2bd[...], preferred_element_type=jnp.float32) + b2f[...],
        0.0, 1.0)
    oe = jnp.dot(h2, W3bd[...], preferred_element_type=jnp.float32) + b3f[...]
    x = jnp.sum(oe * rw, axis=1, keepdims=True)
    psqt = jnp.sum((wps - bps) * rw, axis=1, keepdims=True)
    out[...] = x + psqt * (u - 0.5)


def kernel(us, them, white_indices, white_values, black_indices, black_values,
           psqt_indices, layer_stack_indices, ft_W, ft_b, router_W, router_b,
           router_ls, W1, b1, W2, b2, W3, b3):
    B = us.shape[0]
    idx_all = jnp.concatenate([white_indices, black_indices], axis=0)
    val_all = jnp.concatenate([white_values, black_values], axis=0)

    acc = _ft_bag(ft_W, idx_all.astype(jnp.int32),
                  val_all).reshape(2 * B, D)

    u = jax.random.uniform(jax.random.key(42), (B, NLS),
                           minval=1e-6, maxval=1.0 - 1e-6)
    gnoise = -jnp.log(-jnp.log(u)) / TAU

    L2d = W2.shape[1]
    W1T = W1.reshape(NLS * W1.shape[1], L1).T
    b1f = b1.reshape(1, -1)
    e_ids = jnp.arange(NLS)
    W2bd = jnp.zeros((NLS * W2.shape[2], NLS * L2d), jnp.float32)
    W2bd = W2bd.at[
        (e_ids[:, None, None] * W2.shape[2]
         + jnp.arange(W2.shape[2])[None, :, None]),
        (e_ids[:, None, None] * L2d + jnp.arange(L2d)[None, None, :]),
    ].set(jnp.transpose(W2, (0, 2, 1)))
    b2f = b2.reshape(1, -1)
    W3bd = jnp.zeros((NLS * L2d, NLS), jnp.float32)
    W3bd = W3bd.at[
        (e_ids[:, None] * L2d + jnp.arange(L2d)[None, :]),
        e_ids[:, None],
    ].set(W3[:, 0, :])
    b3f = b3.reshape(1, -1)

    BLK = 512
    nblk = B // BLK
    grid = (nblk,)
    z2 = lambda i: (i, 0)
    out = pl.pallas_call(
        _dense_kernel,
        grid=grid,
        in_specs=[
            pl.BlockSpec((BLK, D), z2),
            pl.BlockSpec((BLK, D), lambda i: (i + nblk, 0)),
            pl.BlockSpec((BLK, 1), z2),
            pl.BlockSpec((BLK, 1), z2),
            pl.BlockSpec((BLK, NLS), z2),
            pl.BlockSpec((1, D), lambda i: (0, 0)),
            pl.BlockSpec((2 * NRF, NLS), lambda i: (0, 0)),
            pl.BlockSpec((1, NLS), lambda i: (0, 0)),
            pl.BlockSpec((1, 1), lambda i: (0, 0)),
            pl.BlockSpec((L1, NLS * 16), lambda i: (0, 0)),
            pl.BlockSpec((1, NLS * 16), lambda i: (0, 0)),
            pl.BlockSpec((NLS * 16, NLS * 32), lambda i: (0, 0)),
            pl.BlockSpec((1, NLS * 32), lambda i: (0, 0)),
            pl.BlockSpec((NLS * 32, NLS), lambda i: (0, 0)),
            pl.BlockSpec((1, NLS), lambda i: (0, 0)),
        ],
        out_specs=pl.BlockSpec((BLK, 1), z2),
        out_shape=jax.ShapeDtypeStruct((B, 1), jnp.float32),
    )(acc, acc, us, them, gnoise, ft_b.reshape(1, D), router_W,
      router_b.reshape(1, NLS), router_ls.reshape(1, 1), W1T, b1f,
      W2bd, b2f, W3bd, b3f)
    return out
